# baseline (speedup 1.0000x reference)
import jax
import jax.numpy as jnp
from jax import lax
from jax.experimental import pallas as pl
from jax.experimental.pallas import tpu as pltpu
from jax.experimental.pallas import tpu_sc as plsc

N = 10000
E = 320000
H = 64
NUM_GRAPHS = 16
BE = 3200
EPS = 1e-16

W1 = 128
KCH = 400
SUB = 80
NSUB = KCH // SUB
EPT = E // 16
NCH = EPT // KCH
NHALF = 5000
NPH = 5120
NROW_T = NPH // 16


def _leaky(x, slope=0.2):
    return jnp.where(x >= 0, x, slope * x)


def _elu(x):
    return jnp.where(x > 0, x, jnp.expm1(x))


def _elu_k(x):
    return jnp.where(x > 0, x, jnp.exp(x) - 1.0)


def _pass1_body(src1, dstl, b1, eattr, h1, a1d, a1s, a1e, zrow,
                out,
                idx0, idx1, idx2, idx3, idx4, srcf, dstf, b1v, eat, rows,
                a1dv, a1sv, a1ev, acc, sem):
    c = lax.axis_index("c")
    s = lax.axis_index("s")
    wid = s
    idxs = [idx0, idx1, idx2, idx3, idx4]
    pltpu.sync_copy(a1d, a1dv)
    pltpu.sync_copy(a1s, a1sv)
    pltpu.sync_copy(a1e, a1ev)
    pltpu.sync_copy(zrow.at[pl.ds(0, NROW_T)],
                    acc.at[pl.ds(s * NROW_T, NROW_T)])
    plsc.subcore_barrier()

    def chunk_body(ci, carry):
        base = wid * EPT + ci * KCH
        descs = [
            pltpu.async_copy(src1.at[pl.ds(base, KCH)], srcf, sem),
            pltpu.async_copy(dstl.at[pl.ds(c * E + base, KCH)], dstf, sem),
            pltpu.async_copy(b1.at[pl.ds(base, KCH)], b1v, sem),
            pltpu.async_copy(eattr.at[pl.ds(base * 16, KCH * 16)], eat, sem),
        ] + [
            pltpu.async_copy(src1.at[pl.ds(base + j * SUB, SUB)], idxs[j], sem)
            for j in range(NSUB)
        ]
        for d in descs:
            d.wait()
        descs = [
            pltpu.async_copy(h1.at[idxs[j]],
                             rows.at[pl.ds(j * SUB, SUB)], sem)
            for j in range(NSUB)
        ]
        for d in descs:
            d.wait()

        def group_body(g, gcarry):
            r0 = g * 16
            srcv = srcf[pl.ds(r0, 16)]
            dstv = dstf[pl.ds(r0, 16)]
            a1dd = plsc.load_gather(a1dv, [dstv])
            a1ss = plsc.load_gather(a1sv, [srcv])
            a1ed = plsc.load_gather(a1ev, [dstv])
            wn = jnp.exp(_leaky(a1dd + a1ss))
            we = jnp.exp(_leaky(a1ed + b1v[pl.ds(r0, 16)]))
            lanes = lax.iota(jnp.int32, 16)
            plsc.store_scatter(rows, [r0 + lanes,
                                      jnp.full((16,), H + 16, jnp.int32)], wn)
            plsc.store_scatter(rows, [r0 + lanes,
                                      jnp.full((16,), H + 17, jnp.int32)], we)
            for k in range(16):
                r = r0 + k
                wnk = wn[k]
                wek = we[k]
                for q in range(H // 16):
                    rows[r, pl.ds(q * 16, 16)] = wnk * rows[r, pl.ds(q * 16, 16)]
                rows[r, pl.ds(H, 16)] = wek * eat[pl.ds(r * 16, 16)]
            return gcarry

        lax.fori_loop(0, KCH // 16, group_body, 0)
        descs = [
            pltpu.async_copy(dstl.at[pl.ds(c * E + base + j * SUB, SUB)],
                             idxs[j], sem)
            for j in range(NSUB)
        ]
        for d in descs:
            d.wait()
        descs = [
            pltpu.async_copy(rows.at[pl.ds(j * SUB, SUB)],
                             acc.at[idxs[j]], sem, add=True)
            for j in range(NSUB)
        ]
        for d in descs:
            d.wait()
        return carry

    lax.fori_loop(0, NCH, chunk_body, 0)
    plsc.subcore_barrier()
    pltpu.sync_copy(acc.at[pl.ds(s * NROW_T, NROW_T)],
                    out.at[c, pl.ds(s * NROW_T, NROW_T)])


def _pass1_call(src, dst, eattr, b1, h1, a1d, a1s, a1e):
    zrow = jnp.zeros((NROW_T, W1), jnp.float32)
    f = pl.kernel(
        _pass1_body,
        out_type=jax.ShapeDtypeStruct((2, NPH, W1), jnp.float32),
        mesh=plsc.VectorSubcoreMesh(core_axis_name="c", subcore_axis_name="s"),
        compiler_params=pltpu.CompilerParams(needs_layout_passes=False),
        scratch_types=[
            pltpu.VMEM((SUB,), jnp.int32),
            pltpu.VMEM((SUB,), jnp.int32),
            pltpu.VMEM((SUB,), jnp.int32),
            pltpu.VMEM((SUB,), jnp.int32),
            pltpu.VMEM((SUB,), jnp.int32),
            pltpu.VMEM((KCH,), jnp.int32),
            pltpu.VMEM((KCH,), jnp.int32),
            pltpu.VMEM((KCH,), jnp.float32),
            pltpu.VMEM((KCH * 16,), jnp.float32),
            pltpu.VMEM((KCH, 128), jnp.float32),
            pltpu.VMEM((N,), jnp.float32),
            pltpu.VMEM((N,), jnp.float32),
            pltpu.VMEM((N,), jnp.float32),
            pltpu.VMEM_SHARED((NPH, W1), jnp.float32),
            pltpu.SemaphoreType.DMA,
        ],
    )
    h1p = jnp.pad(h1, ((0, 0), (0, 128 - H)))
    dl0 = jnp.where(dst < NHALF, dst, NPH - 8)
    dl1 = jnp.where(dst >= NHALF, dst - NHALF, NPH - 8)
    dstl = jnp.concatenate([dl0, dl1])
    parts = f(src, dstl, b1, eattr.reshape(E * 16), h1p, a1d, a1s, a1e, zrow)
    return jnp.concatenate([parts[0, :NHALF], parts[1, :NHALF]], axis=0)


def _gprep_body(eattr_ref, w_ref, out_ref):
    eattr = eattr_ref[...]
    e1We = w_ref[0:16, 0:H]
    wgb = w_ref[16:17, 0:H]
    wce = w_ref[17:18, 0:16]
    g = _elu_k(jnp.dot(eattr, e1We, preferred_element_type=jnp.float32))
    gb = jnp.sum(g * wgb, axis=1, keepdims=True)
    ce = jnp.sum(eattr * wce, axis=1, keepdims=True)
    zero = jnp.zeros((eattr.shape[0], 128 - H - 2), jnp.float32)
    out_ref[...] = jnp.concatenate([g, gb, ce, zero], axis=1)


KA = 160
SA = 80
NSA = KA // SA
NCA = EPT // KA


def _pass2a_body(src1, dstg, dstl, cep, gbp, T, cn, a2d, a2s, a2e, w1p, zrow,
                 outA, wn2o, we2o,
                 ia0, ia1, ib0, ib1, srcf, dstf, cev, gbv, wn2v, we2v,
                 Ts, Td, cnv, a2dv, a2sv, a2ev, w1v, acc, sem):
    c = lax.axis_index("c")
    s = lax.axis_index("s")
    wid = s
    pltpu.sync_copy(w1p, w1v)
    pltpu.sync_copy(cn, cnv)
    pltpu.sync_copy(a2d, a2dv)
    pltpu.sync_copy(a2s, a2sv)
    pltpu.sync_copy(a2e, a2ev)
    pltpu.sync_copy(zrow.at[pl.ds(0, NROW_T)],
                    acc.at[pl.ds(s * NROW_T, NROW_T)])
    plsc.subcore_barrier()
    ias = [ia0, ia1]
    ibs = [ib0, ib1]

    def chunk_body(ci, carry):
        base = wid * EPT + ci * KA
        descs = [
            pltpu.async_copy(src1.at[pl.ds(base, KA)], srcf, sem),
            pltpu.async_copy(dstg.at[pl.ds(base, KA)], dstf, sem),
            pltpu.async_copy(cep.at[pl.ds(base, KA)], cev, sem),
            pltpu.async_copy(gbp.at[pl.ds(base, KA)], gbv, sem),
        ] + [
            pltpu.async_copy(src1.at[pl.ds(base + j * SA, SA)], ias[j], sem)
            for j in range(NSA)
        ] + [
            pltpu.async_copy(dstg.at[pl.ds(base + j * SA, SA)], ibs[j], sem)
            for j in range(NSA)
        ]
        for d in descs:
            d.wait()
        descs = [
            pltpu.async_copy(T.at[ias[j]], Ts.at[pl.ds(j * SA, SA)], sem)
            for j in range(NSA)
        ] + [
            pltpu.async_copy(T.at[ibs[j]], Td.at[pl.ds(j * SA, SA)], sem)
            for j in range(NSA)
        ]
        for d in descs:
            d.wait()

        def group_body(g, gcarry):
            r0 = g * 16
            lanes = lax.iota(jnp.int32, 16)
            rl = r0 + lanes
            srcv = srcf[pl.ds(r0, 16)]
            dstv = dstf[pl.ds(r0, 16)]
            ce = cev[pl.ds(r0, 16)]
            cns = plsc.load_gather(cnv, [srcv])
            cnd = plsc.load_gather(cnv, [dstv])
            ls = _leaky(ce + cns)
            ld = _leaky(ce + cnd)
            m2 = jnp.maximum(ls, ld)
            es = jnp.exp(ls - m2)
            ed = jnp.exp(ld - m2)
            rden = 1.0 / (es + ed + EPS)
            esn = es * rden
            edn = ed * rden
            b2 = gbv[pl.ds(r0, 16)]
            w1q = [w1tab_0, w1tab_1, w1tab_2, w1tab_3]
            for j in range(H):
                jj = jnp.full((16,), j, jnp.int32)
                hns_j = plsc.load_gather(Ts, [rl, jj])
                hnd_j = plsc.load_gather(Td, [rl, jj])
                np_j = esn * hns_j + edn * hnd_j
                enp_j = jnp.where(np_j > 0, np_j, jnp.exp(np_j) - 1.0)
                plsc.store_scatter(Td, [rl, jj], enp_j)
                b2 = b2 + enp_j * w1q[j // 16][j % 16]
            a2dd = plsc.load_gather(a2dv, [dstv])
            a2ss = plsc.load_gather(a2sv, [srcv])
            a2ed = plsc.load_gather(a2ev, [dstv])
            wn2 = jnp.exp(_leaky(a2dd + a2ss))
            we2 = jnp.exp(_leaky(a2ed + b2))
            wn2v[pl.ds(r0, 16)] = wn2
            we2v[pl.ds(r0, 16)] = we2
            for j in range(H):
                jj = jnp.full((16,), j, jnp.int32)
                jh = jnp.full((16,), H + j, jnp.int32)
                h2s_j = plsc.load_gather(Ts, [rl, jh])
                enp_j = plsc.load_gather(Td, [rl, jj])
                plsc.store_scatter(Ts, [rl, jj], wn2 * h2s_j)
                plsc.store_scatter(Ts, [rl, jh], we2 * enp_j)
            return gcarry

        lax.fori_loop(0, KA // 16, group_body, 0)
        descs = []

        @pl.when(c == 0)
        def _exports():
            pltpu.async_copy(wn2v, wn2o.at[pl.ds(base, KA)], sem).wait()
            pltpu.async_copy(we2v, we2o.at[pl.ds(base, KA)], sem).wait()

        descs = [
            pltpu.async_copy(dstl.at[pl.ds(c * E + base + j * SA, SA)],
                             ias[j], sem)
            for j in range(NSA)
        ]
        for d in descs:
            d.wait()
        descs = [
            pltpu.async_copy(Ts.at[pl.ds(j * SA, SA)],
                             acc.at[ias[j]], sem, add=True)
            for j in range(NSA)
        ]
        for d in descs:
            d.wait()
        return carry

    w1tab_0 = w1v[pl.ds(0, 16)]
    w1tab_1 = w1v[pl.ds(16, 16)]
    w1tab_2 = w1v[pl.ds(32, 16)]
    w1tab_3 = w1v[pl.ds(48, 16)]
    lax.fori_loop(0, NCA, chunk_body, 0)
    plsc.subcore_barrier()
    pltpu.sync_copy(acc.at[pl.ds(s * NROW_T, NROW_T)],
                    outA.at[c, pl.ds(s * NROW_T, NROW_T)])


def _pass2b_body(dstl, Gp, wn2i, we2i, zrow,
                 out,
                 idx0, idx1, idx2, idx3, idx4, wnv, wev, rows, acc, sem):
    c = lax.axis_index("c")
    s = lax.axis_index("s")
    wid = s
    idxs = [idx0, idx1, idx2, idx3, idx4]
    pltpu.sync_copy(zrow.at[pl.ds(0, NROW_T)],
                    acc.at[pl.ds(s * NROW_T, NROW_T)])
    plsc.subcore_barrier()

    def chunk_body(ci, carry):
        base = wid * EPT + ci * KCH
        descs = [
            pltpu.async_copy(Gp.at[pl.ds(base, KCH)], rows, sem),
            pltpu.async_copy(wn2i.at[pl.ds(base, KCH)], wnv, sem),
            pltpu.async_copy(we2i.at[pl.ds(base, KCH)], wev, sem),
        ]
        for d in descs:
            d.wait()

        def group_body(g, gcarry):
            r0 = g * 16
            lanes = lax.iota(jnp.int32, 16)
            wn2 = wnv[pl.ds(r0, 16)]
            we2 = wev[pl.ds(r0, 16)]
            plsc.store_scatter(rows, [r0 + lanes,
                                      jnp.full((16,), H, jnp.int32)], wn2)
            plsc.store_scatter(rows, [r0 + lanes,
                                      jnp.full((16,), H + 1, jnp.int32)], we2)
            for k in range(16):
                r = r0 + k
                wek = we2[k]
                for q in range(H // 16):
                    rows[r, pl.ds(q * 16, 16)] = wek * rows[r, pl.ds(q * 16, 16)]
            return gcarry

        lax.fori_loop(0, KCH // 16, group_body, 0)
        descs = [
            pltpu.async_copy(dstl.at[pl.ds(c * E + base + j * SUB, SUB)],
                             idxs[j], sem)
            for j in range(NSUB)
        ]
        for d in descs:
            d.wait()
        descs = [
            pltpu.async_copy(rows.at[pl.ds(j * SUB, SUB)],
                             acc.at[idxs[j]], sem, add=True)
            for j in range(NSUB)
        ]
        for d in descs:
            d.wait()
        return carry

    lax.fori_loop(0, NCH, chunk_body, 0)
    plsc.subcore_barrier()
    pltpu.sync_copy(acc.at[pl.ds(s * NROW_T, NROW_T)],
                    out.at[c, pl.ds(s * NROW_T, NROW_T)])


def _pass2a_call(src, dst, ce, gb, T, cn, a2d, a2s, a2e, w1p):
    zrow = jnp.zeros((NROW_T, W1), jnp.float32)
    f = pl.kernel(
        _pass2a_body,
        out_type=[
            jax.ShapeDtypeStruct((2, NPH, W1), jnp.float32),
            jax.ShapeDtypeStruct((E,), jnp.float32),
            jax.ShapeDtypeStruct((E,), jnp.float32),
        ],
        mesh=plsc.VectorSubcoreMesh(core_axis_name="c", subcore_axis_name="s"),
        compiler_params=pltpu.CompilerParams(needs_layout_passes=False),
        scratch_types=[
            pltpu.VMEM((SA,), jnp.int32),
            pltpu.VMEM((SA,), jnp.int32),
            pltpu.VMEM((SA,), jnp.int32),
            pltpu.VMEM((SA,), jnp.int32),
            pltpu.VMEM((KA,), jnp.int32),
            pltpu.VMEM((KA,), jnp.int32),
            pltpu.VMEM((KA,), jnp.float32),
            pltpu.VMEM((KA,), jnp.float32),
            pltpu.VMEM((KA,), jnp.float32),
            pltpu.VMEM((KA,), jnp.float32),
            pltpu.VMEM((KA, 128), jnp.float32),
            pltpu.VMEM((KA, 128), jnp.float32),
            pltpu.VMEM((N,), jnp.float32),
            pltpu.VMEM((N,), jnp.float32),
            pltpu.VMEM((N,), jnp.float32),
            pltpu.VMEM((N,), jnp.float32),
            pltpu.VMEM((H,), jnp.float32),
            pltpu.VMEM_SHARED((NPH, W1), jnp.float32),
            pltpu.SemaphoreType.DMA,
        ],
    )
    dl0 = jnp.where(dst < NHALF, dst, NPH - 8)
    dl1 = jnp.where(dst >= NHALF, dst - NHALF, NPH - 8)
    dstl = jnp.concatenate([dl0, dl1])
    outA, wn2o, we2o = f(src, dst, dstl, ce, gb, T, cn, a2d, a2s, a2e, w1p,
                         zrow)
    aggA = jnp.concatenate([outA[0, :NHALF], outA[1, :NHALF]], axis=0)
    return aggA, wn2o, we2o


def _pass2b_call(dst, Gp, wn2i, we2i):
    zrow = jnp.zeros((NROW_T, W1), jnp.float32)
    f = pl.kernel(
        _pass2b_body,
        out_type=jax.ShapeDtypeStruct((2, NPH, W1), jnp.float32),
        mesh=plsc.VectorSubcoreMesh(core_axis_name="c", subcore_axis_name="s"),
        compiler_params=pltpu.CompilerParams(needs_layout_passes=False),
        scratch_types=[
            pltpu.VMEM((SUB,), jnp.int32),
            pltpu.VMEM((SUB,), jnp.int32),
            pltpu.VMEM((SUB,), jnp.int32),
            pltpu.VMEM((SUB,), jnp.int32),
            pltpu.VMEM((SUB,), jnp.int32),
            pltpu.VMEM((KCH,), jnp.float32),
            pltpu.VMEM((KCH,), jnp.float32),
            pltpu.VMEM((KCH, 128), jnp.float32),
            pltpu.VMEM_SHARED((NPH, W1), jnp.float32),
            pltpu.SemaphoreType.DMA,
        ],
    )
    dl0 = jnp.where(dst < NHALF, dst, NPH - 8)
    dl1 = jnp.where(dst >= NHALF, dst - NHALF, NPH - 8)
    dstl = jnp.concatenate([dl0, dl1])
    parts = f(dstl, Gp, wn2i, we2i, zrow)
    return jnp.concatenate([parts[0, :NHALF], parts[1, :NHALF]], axis=0)


def _seg_sum(v, seg, n):
    return jax.ops.segment_sum(v, seg, num_segments=n)


def kernel(x, edge_attr, edge_index, batch, n1_Wn, n1_an, n1_We, n1_ae, e1_Wn, e1_We, e1_a, n2_Wn, n2_an, n2_We, n2_ae, Wr, br):
    src, dst = edge_index[0], edge_index[1]
    nb = E // BE
    h1 = x @ n1_Wn
    a1d = h1 @ n1_an[:H]
    a1s = h1 @ n1_an[H:]
    a1e = h1 @ n1_ae[:H]
    b1 = edge_attr @ (n1_We @ n1_ae[H:])

    agg1 = _pass1_call(src, dst, edge_attr, b1, h1, a1d, a1s, a1e)
    den_n = agg1[:, H + 16]
    den_e = agg1[:, H + 17]
    nagg = agg1[:, :H] / (den_n + EPS)[:, None]
    eagg = (agg1[:, H:H + 16] @ n1_We) / (den_e + EPS)[:, None]
    x1 = _elu(jnp.concatenate([nagg, eagg], axis=1))

    hn = x1 @ e1_Wn
    h2 = x1 @ n2_Wn
    ae2v = n2_ae[H:]
    W_np, W_g = n2_We[:H], n2_We[H:]
    T = jnp.concatenate([hn, h2], axis=1)
    cn = hn @ e1_a[H:]
    a2d = h2 @ n2_an[:H]
    a2s = h2 @ n2_an[H:]
    a2e = h2 @ n2_ae[:H]
    w1p = W_np @ ae2v

    gpk = jnp.zeros((24, 128), jnp.float32)
    gpk = gpk.at[0:16, 0:H].set(e1_We)
    gpk = gpk.at[16, 0:H].set(W_g @ ae2v)
    gpk = gpk.at[17, 0:16].set(e1_We @ e1_a[:H])
    espec = pl.BlockSpec((BE, 16), lambda i: (i, 0))
    Gp = pl.pallas_call(
        _gprep_body,
        grid=(nb,),
        in_specs=[espec, pl.BlockSpec((24, 128), lambda i: (0, 0))],
        out_specs=pl.BlockSpec((BE, 128), lambda i: (i, 0)),
        out_shape=jax.ShapeDtypeStruct((E, 128), jnp.float32),
    )(edge_attr, gpk)
    gb = Gp[:, H]
    ce = Gp[:, H + 1]

    aggA, wn2o, we2o = _pass2a_call(src, dst, ce, gb, T, cn, a2d, a2s, a2e,
                                    w1p)
    aggB = _pass2b_call(dst, Gp, wn2o, we2o)
    den_n2 = aggB[:, H]
    den_e2 = aggB[:, H + 1]
    nagg2 = aggA[:, :H] / (den_n2 + EPS)[:, None]
    eagg2 = (aggA[:, H:2 * H] @ W_np + aggB[:, :H] @ W_g) / (
        den_e2 + EPS)[:, None]
    x2 = _elu(jnp.concatenate([nagg2, eagg2], axis=1))

    sums = _seg_sum(x2, batch, NUM_GRAPHS)
    cnts = _seg_sum(jnp.ones((N,), jnp.float32), batch, NUM_GRAPHS)
    gpool = sums / jnp.maximum(cnts, 1.0)[:, None]
    return gpool @ Wr + br


# speedup vs baseline: 3.2222x; 3.2222x over previous
import jax
import jax.numpy as jnp
from jax import lax
from jax.experimental import pallas as pl
from jax.experimental.pallas import tpu as pltpu
from jax.experimental.pallas import tpu_sc as plsc

N = 10000
E = 320000
H = 64
NUM_GRAPHS = 16
BE = 3200
EPS = 1e-16

W1 = 128
KCH = 400
SUB = 80
NSUB = KCH // SUB
EPT = E // 16
NCH = EPT // KCH
NHALF = 5000
NPH = 5120
NROW_T = NPH // 16


def _leaky(x, slope=0.2):
    return jnp.where(x >= 0, x, slope * x)


def _elu(x):
    return jnp.where(x > 0, x, jnp.expm1(x))


def _elu_k(x):
    return jnp.where(x > 0, x, jnp.exp(x) - 1.0)


def _pass1_body(src1, dstl, b1, eattr, h1, a1d, a1s, a1e, zrow,
                out,
                idx0, idx1, idx2, idx3, idx4, srcf, dstf, b1v, eat, rows,
                a1dv, a1sv, a1ev, acc, sem):
    c = lax.axis_index("c")
    s = lax.axis_index("s")
    wid = s
    idxs = [idx0, idx1, idx2, idx3, idx4]
    pltpu.sync_copy(a1d, a1dv)
    pltpu.sync_copy(a1s, a1sv)
    pltpu.sync_copy(a1e, a1ev)
    pltpu.sync_copy(zrow.at[pl.ds(0, NROW_T)],
                    acc.at[pl.ds(s * NROW_T, NROW_T)])
    plsc.subcore_barrier()

    def chunk_body(ci, carry):
        base = wid * EPT + ci * KCH
        descs = [
            pltpu.async_copy(src1.at[pl.ds(base, KCH)], srcf, sem),
            pltpu.async_copy(dstl.at[pl.ds(c * E + base, KCH)], dstf, sem),
            pltpu.async_copy(b1.at[pl.ds(base, KCH)], b1v, sem),
            pltpu.async_copy(eattr.at[pl.ds(base * 16, KCH * 16)], eat, sem),
        ] + [
            pltpu.async_copy(src1.at[pl.ds(base + j * SUB, SUB)], idxs[j], sem)
            for j in range(NSUB)
        ]
        for d in descs:
            d.wait()
        descs = [
            pltpu.async_copy(h1.at[idxs[j]],
                             rows.at[pl.ds(j * SUB, SUB)], sem)
            for j in range(NSUB)
        ]
        for d in descs:
            d.wait()

        def group_body(g, gcarry):
            r0 = g * 16
            srcv = srcf[pl.ds(r0, 16)]
            dstv = dstf[pl.ds(r0, 16)]
            a1dd = plsc.load_gather(a1dv, [dstv])
            a1ss = plsc.load_gather(a1sv, [srcv])
            a1ed = plsc.load_gather(a1ev, [dstv])
            wn = jnp.exp(_leaky(a1dd + a1ss))
            we = jnp.exp(_leaky(a1ed + b1v[pl.ds(r0, 16)]))
            lanes = lax.iota(jnp.int32, 16)
            plsc.store_scatter(rows, [r0 + lanes,
                                      jnp.full((16,), H + 16, jnp.int32)], wn)
            plsc.store_scatter(rows, [r0 + lanes,
                                      jnp.full((16,), H + 17, jnp.int32)], we)
            for k in range(16):
                r = r0 + k
                wnk = wn[k]
                wek = we[k]
                for q in range(H // 16):
                    rows[r, pl.ds(q * 16, 16)] = wnk * rows[r, pl.ds(q * 16, 16)]
                rows[r, pl.ds(H, 16)] = wek * eat[pl.ds(r * 16, 16)]
            return gcarry

        lax.fori_loop(0, KCH // 16, group_body, 0)
        descs = [
            pltpu.async_copy(dstl.at[pl.ds(c * E + base + j * SUB, SUB)],
                             idxs[j], sem)
            for j in range(NSUB)
        ]
        for d in descs:
            d.wait()
        descs = [
            pltpu.async_copy(rows.at[pl.ds(j * SUB, SUB)],
                             acc.at[idxs[j]], sem, add=True)
            for j in range(NSUB)
        ]
        for d in descs:
            d.wait()
        return carry

    lax.fori_loop(0, NCH, chunk_body, 0)
    plsc.subcore_barrier()
    pltpu.sync_copy(acc.at[pl.ds(s * NROW_T, NROW_T)],
                    out.at[c, pl.ds(s * NROW_T, NROW_T)])


def _pass1_call(src, dst, eattr, b1, h1, a1d, a1s, a1e):
    zrow = jnp.zeros((NROW_T, W1), jnp.float32)
    f = pl.kernel(
        _pass1_body,
        out_type=jax.ShapeDtypeStruct((2, NPH, W1), jnp.float32),
        mesh=plsc.VectorSubcoreMesh(core_axis_name="c", subcore_axis_name="s"),
        compiler_params=pltpu.CompilerParams(needs_layout_passes=False),
        scratch_types=[
            pltpu.VMEM((SUB,), jnp.int32),
            pltpu.VMEM((SUB,), jnp.int32),
            pltpu.VMEM((SUB,), jnp.int32),
            pltpu.VMEM((SUB,), jnp.int32),
            pltpu.VMEM((SUB,), jnp.int32),
            pltpu.VMEM((KCH,), jnp.int32),
            pltpu.VMEM((KCH,), jnp.int32),
            pltpu.VMEM((KCH,), jnp.float32),
            pltpu.VMEM((KCH * 16,), jnp.float32),
            pltpu.VMEM((KCH, 128), jnp.float32),
            pltpu.VMEM((N,), jnp.float32),
            pltpu.VMEM((N,), jnp.float32),
            pltpu.VMEM((N,), jnp.float32),
            pltpu.VMEM_SHARED((NPH, W1), jnp.float32),
            pltpu.SemaphoreType.DMA,
        ],
    )
    h1p = jnp.pad(h1, ((0, 0), (0, 128 - H)))
    dl0 = jnp.where(dst < NHALF, dst, NPH - 8)
    dl1 = jnp.where(dst >= NHALF, dst - NHALF, NPH - 8)
    dstl = jnp.concatenate([dl0, dl1])
    parts = f(src, dstl, b1, eattr.reshape(E * 16), h1p, a1d, a1s, a1e, zrow)
    return jnp.concatenate([parts[0, :NHALF], parts[1, :NHALF]], axis=0)


def _gprep_body(eattr_ref, w_ref, out_ref):
    eattr = eattr_ref[...]
    e1We = w_ref[0:16, 0:H]
    wgb = w_ref[16:17, 0:H]
    wce = w_ref[17:18, 0:16]
    g = _elu_k(jnp.dot(eattr, e1We, preferred_element_type=jnp.float32))
    gb = jnp.sum(g * wgb, axis=1, keepdims=True)
    ce = jnp.sum(eattr * wce, axis=1, keepdims=True)
    zero = jnp.zeros((eattr.shape[0], 128 - H - 2), jnp.float32)
    out_ref[...] = jnp.concatenate([g, gb, ce, zero], axis=1)


KA = 160
SA = 80
NSA = KA // SA
NCA = EPT // KA


def _pass2a_body(src1, dstg, dstl, cep, gbp, T, cn, a2d, a2s, a2e, w1p, zrow,
                 outA, wn2o, we2o,
                 ia0, ia1, ib0, ib1, srcf, dstf, cev, gbv, wn2v, we2v,
                 Ts, Td, cnv, a2dv, a2sv, a2ev, w1v, pbuf, acc, sem):
    c = lax.axis_index("c")
    s = lax.axis_index("s")
    wid = s
    pltpu.sync_copy(w1p, w1v)
    pltpu.sync_copy(cn, cnv)
    pltpu.sync_copy(a2d, a2dv)
    pltpu.sync_copy(a2s, a2sv)
    pltpu.sync_copy(a2e, a2ev)
    pltpu.sync_copy(zrow.at[pl.ds(0, NROW_T)],
                    acc.at[pl.ds(s * NROW_T, NROW_T)])
    plsc.subcore_barrier()
    ias = [ia0, ia1]
    ibs = [ib0, ib1]

    def chunk_body(ci, carry):
        base = wid * EPT + ci * KA
        descs = [
            pltpu.async_copy(src1.at[pl.ds(base, KA)], srcf, sem),
            pltpu.async_copy(dstg.at[pl.ds(base, KA)], dstf, sem),
            pltpu.async_copy(cep.at[pl.ds(base, KA)], cev, sem),
            pltpu.async_copy(gbp.at[pl.ds(base, KA)], gbv, sem),
        ] + [
            pltpu.async_copy(src1.at[pl.ds(base + j * SA, SA)], ias[j], sem)
            for j in range(NSA)
        ] + [
            pltpu.async_copy(dstg.at[pl.ds(base + j * SA, SA)], ibs[j], sem)
            for j in range(NSA)
        ]
        for d in descs:
            d.wait()
        descs = [
            pltpu.async_copy(T.at[ias[j]], Ts.at[pl.ds(j * SA, SA)], sem)
            for j in range(NSA)
        ] + [
            pltpu.async_copy(T.at[ibs[j]], Td.at[pl.ds(j * SA, SA)], sem)
            for j in range(NSA)
        ]
        for d in descs:
            d.wait()

        def group_body(g, gcarry):
            r0 = g * 16
            lanes = lax.iota(jnp.int32, 16)
            srcv = srcf[pl.ds(r0, 16)]
            dstv = dstf[pl.ds(r0, 16)]
            ce = cev[pl.ds(r0, 16)]
            cns = plsc.load_gather(cnv, [srcv])
            cnd = plsc.load_gather(cnv, [dstv])
            ls = _leaky(ce + cns)
            ld = _leaky(ce + cnd)
            m2 = jnp.maximum(ls, ld)
            es = jnp.exp(ls - m2)
            ed = jnp.exp(ld - m2)
            rden = 1.0 / (es + ed + EPS)
            esn = es * rden
            edn = ed * rden
            w1q = [w1tab_0, w1tab_1, w1tab_2, w1tab_3]
            for k in range(16):
                r = r0 + k
                esk = esn[k]
                edk = edn[k]
                p = jnp.zeros((16,), jnp.float32)
                for q in range(H // 16):
                    hns_q = Ts[r, pl.ds(q * 16, 16)]
                    hnd_q = Td[r, pl.ds(q * 16, 16)]
                    np_q = esk * hns_q + edk * hnd_q
                    enp_q = jnp.where(np_q > 0, np_q, jnp.exp(np_q) - 1.0)
                    Td[r, pl.ds(q * 16, 16)] = enp_q
                    p = p + enp_q * w1q[q]
                pbuf[pl.ds(k * 16, 16)] = p
            b2 = gbv[pl.ds(r0, 16)]
            l16 = lanes * 16
            for j in range(16):
                b2 = b2 + plsc.load_gather(pbuf, [l16 + j])
            a2dd = plsc.load_gather(a2dv, [dstv])
            a2ss = plsc.load_gather(a2sv, [srcv])
            a2ed = plsc.load_gather(a2ev, [dstv])
            wn2 = jnp.exp(_leaky(a2dd + a2ss))
            we2 = jnp.exp(_leaky(a2ed + b2))
            wn2v[pl.ds(r0, 16)] = wn2
            we2v[pl.ds(r0, 16)] = we2
            for k in range(16):
                r = r0 + k
                wnk = wn2[k]
                wek = we2[k]
                for q in range(H // 16):
                    h2s_q = Ts[r, pl.ds(H + q * 16, 16)]
                    enp_q = Td[r, pl.ds(q * 16, 16)]
                    Ts[r, pl.ds(q * 16, 16)] = wnk * h2s_q
                    Ts[r, pl.ds(H + q * 16, 16)] = wek * enp_q
            return gcarry

        lax.fori_loop(0, KA // 16, group_body, 0)
        descs = []

        @pl.when(c == 0)
        def _exports():
            pltpu.async_copy(wn2v, wn2o.at[pl.ds(base, KA)], sem).wait()
            pltpu.async_copy(we2v, we2o.at[pl.ds(base, KA)], sem).wait()

        descs = [
            pltpu.async_copy(dstl.at[pl.ds(c * E + base + j * SA, SA)],
                             ias[j], sem)
            for j in range(NSA)
        ]
        for d in descs:
            d.wait()
        descs = [
            pltpu.async_copy(Ts.at[pl.ds(j * SA, SA)],
                             acc.at[ias[j]], sem, add=True)
            for j in range(NSA)
        ]
        for d in descs:
            d.wait()
        return carry

    w1tab_0 = w1v[pl.ds(0, 16)]
    w1tab_1 = w1v[pl.ds(16, 16)]
    w1tab_2 = w1v[pl.ds(32, 16)]
    w1tab_3 = w1v[pl.ds(48, 16)]
    lax.fori_loop(0, NCA, chunk_body, 0)
    plsc.subcore_barrier()
    pltpu.sync_copy(acc.at[pl.ds(s * NROW_T, NROW_T)],
                    outA.at[c, pl.ds(s * NROW_T, NROW_T)])


def _pass2b_body(dstl, Gp, wn2i, we2i, zrow,
                 out,
                 idx0, idx1, idx2, idx3, idx4, wnv, wev, rows, acc, sem):
    c = lax.axis_index("c")
    s = lax.axis_index("s")
    wid = s
    idxs = [idx0, idx1, idx2, idx3, idx4]
    pltpu.sync_copy(zrow.at[pl.ds(0, NROW_T)],
                    acc.at[pl.ds(s * NROW_T, NROW_T)])
    plsc.subcore_barrier()

    def chunk_body(ci, carry):
        base = wid * EPT + ci * KCH
        descs = [
            pltpu.async_copy(Gp.at[pl.ds(base, KCH)], rows, sem),
            pltpu.async_copy(wn2i.at[pl.ds(base, KCH)], wnv, sem),
            pltpu.async_copy(we2i.at[pl.ds(base, KCH)], wev, sem),
        ]
        for d in descs:
            d.wait()

        def group_body(g, gcarry):
            r0 = g * 16
            lanes = lax.iota(jnp.int32, 16)
            wn2 = wnv[pl.ds(r0, 16)]
            we2 = wev[pl.ds(r0, 16)]
            plsc.store_scatter(rows, [r0 + lanes,
                                      jnp.full((16,), H, jnp.int32)], wn2)
            plsc.store_scatter(rows, [r0 + lanes,
                                      jnp.full((16,), H + 1, jnp.int32)], we2)
            for k in range(16):
                r = r0 + k
                wek = we2[k]
                for q in range(H // 16):
                    rows[r, pl.ds(q * 16, 16)] = wek * rows[r, pl.ds(q * 16, 16)]
            return gcarry

        lax.fori_loop(0, KCH // 16, group_body, 0)
        descs = [
            pltpu.async_copy(dstl.at[pl.ds(c * E + base + j * SUB, SUB)],
                             idxs[j], sem)
            for j in range(NSUB)
        ]
        for d in descs:
            d.wait()
        descs = [
            pltpu.async_copy(rows.at[pl.ds(j * SUB, SUB)],
                             acc.at[idxs[j]], sem, add=True)
            for j in range(NSUB)
        ]
        for d in descs:
            d.wait()
        return carry

    lax.fori_loop(0, NCH, chunk_body, 0)
    plsc.subcore_barrier()
    pltpu.sync_copy(acc.at[pl.ds(s * NROW_T, NROW_T)],
                    out.at[c, pl.ds(s * NROW_T, NROW_T)])


def _pass2a_call(src, dst, ce, gb, T, cn, a2d, a2s, a2e, w1p):
    zrow = jnp.zeros((NROW_T, W1), jnp.float32)
    f = pl.kernel(
        _pass2a_body,
        out_type=[
            jax.ShapeDtypeStruct((2, NPH, W1), jnp.float32),
            jax.ShapeDtypeStruct((E,), jnp.float32),
            jax.ShapeDtypeStruct((E,), jnp.float32),
        ],
        mesh=plsc.VectorSubcoreMesh(core_axis_name="c", subcore_axis_name="s"),
        compiler_params=pltpu.CompilerParams(needs_layout_passes=False),
        scratch_types=[
            pltpu.VMEM((SA,), jnp.int32),
            pltpu.VMEM((SA,), jnp.int32),
            pltpu.VMEM((SA,), jnp.int32),
            pltpu.VMEM((SA,), jnp.int32),
            pltpu.VMEM((KA,), jnp.int32),
            pltpu.VMEM((KA,), jnp.int32),
            pltpu.VMEM((KA,), jnp.float32),
            pltpu.VMEM((KA,), jnp.float32),
            pltpu.VMEM((KA,), jnp.float32),
            pltpu.VMEM((KA,), jnp.float32),
            pltpu.VMEM((KA, 128), jnp.float32),
            pltpu.VMEM((KA, 128), jnp.float32),
            pltpu.VMEM((N,), jnp.float32),
            pltpu.VMEM((N,), jnp.float32),
            pltpu.VMEM((N,), jnp.float32),
            pltpu.VMEM((N,), jnp.float32),
            pltpu.VMEM((H,), jnp.float32),
            pltpu.VMEM((256,), jnp.float32),
            pltpu.VMEM_SHARED((NPH, W1), jnp.float32),
            pltpu.SemaphoreType.DMA,
        ],
    )
    dl0 = jnp.where(dst < NHALF, dst, NPH - 8)
    dl1 = jnp.where(dst >= NHALF, dst - NHALF, NPH - 8)
    dstl = jnp.concatenate([dl0, dl1])
    outA, wn2o, we2o = f(src, dst, dstl, ce, gb, T, cn, a2d, a2s, a2e, w1p,
                         zrow)
    aggA = jnp.concatenate([outA[0, :NHALF], outA[1, :NHALF]], axis=0)
    return aggA, wn2o, we2o


def _pass2b_call(dst, Gp, wn2i, we2i):
    zrow = jnp.zeros((NROW_T, W1), jnp.float32)
    f = pl.kernel(
        _pass2b_body,
        out_type=jax.ShapeDtypeStruct((2, NPH, W1), jnp.float32),
        mesh=plsc.VectorSubcoreMesh(core_axis_name="c", subcore_axis_name="s"),
        compiler_params=pltpu.CompilerParams(needs_layout_passes=False),
        scratch_types=[
            pltpu.VMEM((SUB,), jnp.int32),
            pltpu.VMEM((SUB,), jnp.int32),
            pltpu.VMEM((SUB,), jnp.int32),
            pltpu.VMEM((SUB,), jnp.int32),
            pltpu.VMEM((SUB,), jnp.int32),
            pltpu.VMEM((KCH,), jnp.float32),
            pltpu.VMEM((KCH,), jnp.float32),
            pltpu.VMEM((KCH, 128), jnp.float32),
            pltpu.VMEM_SHARED((NPH, W1), jnp.float32),
            pltpu.SemaphoreType.DMA,
        ],
    )
    dl0 = jnp.where(dst < NHALF, dst, NPH - 8)
    dl1 = jnp.where(dst >= NHALF, dst - NHALF, NPH - 8)
    dstl = jnp.concatenate([dl0, dl1])
    parts = f(dstl, Gp, wn2i, we2i, zrow)
    return jnp.concatenate([parts[0, :NHALF], parts[1, :NHALF]], axis=0)


def _seg_sum(v, seg, n):
    return jax.ops.segment_sum(v, seg, num_segments=n)


def kernel(x, edge_attr, edge_index, batch, n1_Wn, n1_an, n1_We, n1_ae, e1_Wn, e1_We, e1_a, n2_Wn, n2_an, n2_We, n2_ae, Wr, br):
    src, dst = edge_index[0], edge_index[1]
    nb = E // BE
    h1 = x @ n1_Wn
    a1d = h1 @ n1_an[:H]
    a1s = h1 @ n1_an[H:]
    a1e = h1 @ n1_ae[:H]
    b1 = edge_attr @ (n1_We @ n1_ae[H:])

    agg1 = _pass1_call(src, dst, edge_attr, b1, h1, a1d, a1s, a1e)
    den_n = agg1[:, H + 16]
    den_e = agg1[:, H + 17]
    nagg = agg1[:, :H] / (den_n + EPS)[:, None]
    eagg = (agg1[:, H:H + 16] @ n1_We) / (den_e + EPS)[:, None]
    x1 = _elu(jnp.concatenate([nagg, eagg], axis=1))

    hn = x1 @ e1_Wn
    h2 = x1 @ n2_Wn
    ae2v = n2_ae[H:]
    W_np, W_g = n2_We[:H], n2_We[H:]
    T = jnp.concatenate([hn, h2], axis=1)
    cn = hn @ e1_a[H:]
    a2d = h2 @ n2_an[:H]
    a2s = h2 @ n2_an[H:]
    a2e = h2 @ n2_ae[:H]
    w1p = W_np @ ae2v

    gpk = jnp.zeros((24, 128), jnp.float32)
    gpk = gpk.at[0:16, 0:H].set(e1_We)
    gpk = gpk.at[16, 0:H].set(W_g @ ae2v)
    gpk = gpk.at[17, 0:16].set(e1_We @ e1_a[:H])
    espec = pl.BlockSpec((BE, 16), lambda i: (i, 0))
    Gp = pl.pallas_call(
        _gprep_body,
        grid=(nb,),
        in_specs=[espec, pl.BlockSpec((24, 128), lambda i: (0, 0))],
        out_specs=pl.BlockSpec((BE, 128), lambda i: (i, 0)),
        out_shape=jax.ShapeDtypeStruct((E, 128), jnp.float32),
    )(edge_attr, gpk)
    gb = Gp[:, H]
    ce = Gp[:, H + 1]

    aggA, wn2o, we2o = _pass2a_call(src, dst, ce, gb, T, cn, a2d, a2s, a2e,
                                    w1p)
    aggB = _pass2b_call(dst, Gp, wn2o, we2o)
    den_n2 = aggB[:, H]
    den_e2 = aggB[:, H + 1]
    nagg2 = aggA[:, :H] / (den_n2 + EPS)[:, None]
    eagg2 = (aggA[:, H:2 * H] @ W_np + aggB[:, :H] @ W_g) / (
        den_e2 + EPS)[:, None]
    x2 = _elu(jnp.concatenate([nagg2, eagg2], axis=1))

    sums = _seg_sum(x2, batch, NUM_GRAPHS)
    cnts = _seg_sum(jnp.ones((N,), jnp.float32), batch, NUM_GRAPHS)
    gpool = sums / jnp.maximum(cnts, 1.0)[:, None]
    return gpool @ Wr + br
